# baseline (speedup 1.0000x reference)
import functools

import jax
import jax.numpy as jnp
from jax import lax
from jax.experimental import pallas as pl
from jax.experimental.pallas import tpu as pltpu
from jax.experimental.pallas import tpu_sc as plsc

N_V = 2048
N_F = 100000
F_PAD = 102400
TILE_F = 3200
WIN_F = 400
N_WIN_F = TILE_F // WIN_F
N_ENT = F_PAD * 9
TILE_ENT = N_ENT // 16
WIN_E = 3200
N_WIN_E = TILE_ENT // WIN_E
N_BATCH = WIN_E // 128
Q_SIZE = 512 * 2048
STRIPE = Q_SIZE // 16

_MESH = plsc.VectorSubcoreMesh(
    core_axis_name="c", subcore_axis_name="s", num_cores=2, num_subcores=16)


def _rsqrt(q):
    qc = jnp.maximum(q, jnp.float32(1e-30))
    ib = plsc.bitcast(qc, jnp.int32)
    y = plsc.bitcast(jnp.int32(0x5F3759DF) - (ib >> 1), jnp.float32)
    y = y * (jnp.float32(1.5) - jnp.float32(0.5) * qc * y * y)
    y = y * (jnp.float32(1.5) - jnp.float32(0.5) * qc * y * y)
    return y


def _fused_body(verts_hbm, faces_hbm, out_hbm,
                d_hbm, v_hbm, vtab, fwin, dstage, vstage,
                dwin0, dwin1, vwin0, vwin1,
                offst0, offst1, valst0, valst1, zbuf, slab,
                semin, semstr, semz):
    cc = lax.axis_index("c")
    ss = lax.axis_index("s")
    lane = lax.iota(jnp.int32, 16)
    dwin = (dwin0, dwin1)
    vwin = (vwin0, vwin1)
    offst = (offst0, offst1)
    valst = (valst0, valst1)

    pltpu.sync_copy(verts_hbm, vtab)
    dbase_sc = cc * N_ENT
    vbase_sc = cc * (5 * N_ENT)

    def awin_body(w, _):
        fbase = ss * (F_PAD // 16) + w * WIN_F
        pltpu.sync_copy(faces_hbm.at[pl.ds(fbase * 3, WIN_F * 3)], fwin)

        def vec_body(i, _):
            lf = i * 16 + lane
            f0 = plsc.load_gather(fwin, [lf * 3])
            f1 = plsc.load_gather(fwin, [lf * 3 + 1])
            f2 = plsc.load_gather(fwin, [lf * 3 + 2])
            x0 = plsc.load_gather(vtab, [f0 * 3])
            y0 = plsc.load_gather(vtab, [f0 * 3 + 1])
            z0 = plsc.load_gather(vtab, [f0 * 3 + 2])
            x1 = plsc.load_gather(vtab, [f1 * 3])
            y1 = plsc.load_gather(vtab, [f1 * 3 + 1])
            z1 = plsc.load_gather(vtab, [f1 * 3 + 2])
            x2 = plsc.load_gather(vtab, [f2 * 3])
            y2 = plsc.load_gather(vtab, [f2 * 3 + 1])
            z2 = plsc.load_gather(vtab, [f2 * 3 + 2])

            ax, ay, az = x1 - x0, y1 - y0, z1 - z0
            bx, by, bz = x2 - x0, y2 - y0, z2 - z0
            gx, gy, gz = x2 - x1, y2 - y1, z2 - z1
            cxv = ay * bz - az * by
            cyv = az * bx - ax * bz
            czv = ax * by - ay * bx
            q = cxv * cxv + cyv * cyv + czv * czv
            inv = _rsqrt(q)
            d0 = ax * bx + ay * by + az * bz
            d1 = gx * (-ax) + gy * (-ay) + gz * (-az)
            d2 = bx * gx + by * gy + bz * gz
            c0 = d0 * inv
            c1 = d1 * inv
            c2 = d2 * inv
            area = jnp.float32(0.5) * q * inv

            h = jnp.float32(0.5)
            svals = (h * (c1 + c2), h * (c0 + c2), h * (c0 + c1),
                     -h * c2, -h * c2, -h * c0, -h * c0, -h * c1, -h * c1)
            d6 = area * jnp.float32(1.0 / 6.0)
            o12 = area * jnp.float32(1.0 / 12.0)
            mvals = (d6, d6, d6, o12, o12, o12, o12, o12, o12)
            a30 = area * jnp.float32(1.0 / 30.0)
            a60 = area * jnp.float32(1.0 / 60.0)

            rows = (f0, f1, f2, f0, f1, f1, f2, f2, f0)
            cols = (f0, f1, f2, f1, f0, f2, f1, f0, f2)
            gf = ss * (F_PAD // 16) + w * WIN_F + lf
            real = gf < N_F
            eb = lf * 9

            pos = []
            for u0, u1, u2 in ((x0, x1, x2), (y0, y1, y2), (z0, z1, z2)):
                v00 = a30 * (jnp.float32(3.0) * u0 + u1 + u2)
                v11 = a30 * (u0 + jnp.float32(3.0) * u1 + u2)
                v22 = a30 * (u0 + u1 + jnp.float32(3.0) * u2)
                v01 = a60 * (jnp.float32(2.0) * (u0 + u1) + u2)
                v12 = a60 * (u0 + jnp.float32(2.0) * (u1 + u2))
                v02 = a60 * (jnp.float32(2.0) * (u0 + u2) + u1)
                pos.append((v00, v11, v22, v01, v01, v12, v12, v02, v02))

            zero = jnp.float32(0.0)
            for k in range(9):
                dk = jnp.where(real, rows[k] * 2048 + cols[k], gf * 9 + k)
                plsc.store_scatter(dstage, [eb + k], dk)
                allv = (svals[k], mvals[k], pos[0][k], pos[1][k], pos[2][k])
                for m in range(5):
                    vk = jnp.where(real, allv[m], zero)
                    plsc.store_scatter(vstage, [m * (WIN_F * 9) + eb + k], vk)
            return 0

        lax.fori_loop(0, WIN_F // 16, vec_body, 0)
        pltpu.sync_copy(dstage,
                        d_hbm.at[pl.ds(dbase_sc + fbase * 9, WIN_F * 9)])
        for m in range(5):
            pltpu.sync_copy(
                vstage.at[pl.ds(m * (WIN_F * 9), WIN_F * 9)],
                v_hbm.at[pl.ds(vbase_sc + m * N_ENT + fbase * 9, WIN_F * 9)])
        return 0

    lax.fori_loop(0, (F_PAD // 16) // WIN_F, awin_body, 0)
    plsc.subcore_barrier()

    ebase = ss * TILE_ENT

    def zinit(i, _):
        zbuf[pl.ds(i * 16, 16)] = jnp.zeros((16,), jnp.float32)
        return 0
    lax.fori_loop(0, 4096 // 16, zinit, 0)

    def issue_in(m, w, b):
        pltpu.async_copy(
            d_hbm.at[pl.ds(dbase_sc + ebase + w * WIN_E, WIN_E)],
            dwin[b], semin)
        pltpu.async_copy(
            v_hbm.at[pl.ds(vbase_sc + m * N_ENT + ebase + w * WIN_E, WIN_E)],
            vwin[b], semin)

    def wait_in(m, w, b):
        pltpu.make_async_copy(
            d_hbm.at[pl.ds(dbase_sc + ebase + w * WIN_E, WIN_E)],
            dwin[b], semin).wait()
        pltpu.make_async_copy(
            v_hbm.at[pl.ds(vbase_sc + m * N_ENT + ebase + w * WIN_E, WIN_E)],
            vwin[b], semin).wait()

    def fire_streams(b):
        for j in range(N_BATCH):
            pltpu.async_copy(valst[b].at[j], slab.at[offst[b].at[j]], semstr,
                             add=True)

    def drain_streams(b):
        for j in range(N_BATCH):
            pltpu.make_async_copy(valst[b].at[j], slab.at[offst[b].at[j]],
                                  semstr).wait()

    def pass_body(p, _):
        m = p // 2
        quarter = (p % 2) * 2 + cc
        lo = quarter * Q_SIZE

        zcps = [pltpu.async_copy(
            zbuf, slab.at[pl.ds(ss * STRIPE + j * 4096, 4096)], semz)
            for j in range(STRIPE // 4096)]
        for cp in zcps:
            cp.wait()
        plsc.subcore_barrier()

        issue_in(m, 0, 0)
        issue_in(m, 1, 1)

        def gpair(g, _):
            for b in range(2):
                w = g * 2 + b
                wait_in(m, w, b)

                @pl.when(w >= 2)
                def _():
                    drain_streams(b)

                def vec_body(i, _, b=b):
                    dd = dwin[b][pl.ds(i * 16, 16)]
                    vv = vwin[b][pl.ds(i * 16, 16)]
                    dq = dd - lo
                    msk = (dq >= 0) & (dq < Q_SIZE)
                    off = jnp.where(msk, dq, w * WIN_E + i * 16 + lane)
                    val = jnp.where(msk, vv, jnp.float32(0.0))
                    offst[b][i >> 3, pl.ds((i & 7) * 16, 16)] = off
                    valst[b][i >> 3, pl.ds((i & 7) * 16, 16)] = val
                    return 0

                lax.fori_loop(0, WIN_E // 16, vec_body, 0)

                @pl.when(w + 2 < N_WIN_E)
                def _():
                    issue_in(m, w + 2, b)

                fire_streams(b)
            return 0

        lax.fori_loop(0, N_WIN_E // 2, gpair, 0)
        drain_streams(0)
        drain_streams(1)
        plsc.subcore_barrier()
        pltpu.sync_copy(
            slab.at[pl.ds(ss * STRIPE, STRIPE)],
            out_hbm.at[pl.ds(m * (2048 * 2048) + quarter * Q_SIZE + ss * STRIPE,
                             STRIPE)])
        return 0

    lax.fori_loop(0, 10, pass_body, 0)


_fused_call = pl.kernel(
    _fused_body,
    out_type=jax.ShapeDtypeStruct((5 * 2048 * 2048,), jnp.float32),
    mesh=_MESH,
    compiler_params=pltpu.CompilerParams(needs_layout_passes=False),
    scratch_types=[
        pltpu.HBM((2 * N_ENT,), jnp.int32),
        pltpu.HBM((2 * 5 * N_ENT,), jnp.float32),
        pltpu.VMEM((N_V * 3,), jnp.float32),
        pltpu.VMEM((WIN_F * 3,), jnp.int32),
        pltpu.VMEM((WIN_F * 9,), jnp.int32),
        pltpu.VMEM((5 * WIN_F * 9,), jnp.float32),
        pltpu.VMEM((WIN_E,), jnp.int32),
        pltpu.VMEM((WIN_E,), jnp.int32),
        pltpu.VMEM((WIN_E,), jnp.float32),
        pltpu.VMEM((WIN_E,), jnp.float32),
        pltpu.VMEM((N_BATCH, 128), jnp.int32),
        pltpu.VMEM((N_BATCH, 128), jnp.int32),
        pltpu.VMEM((N_BATCH, 128), jnp.float32),
        pltpu.VMEM((N_BATCH, 128), jnp.float32),
        pltpu.VMEM((4096,), jnp.float32),
        pltpu.VMEM_SHARED((Q_SIZE,), jnp.float32),
        pltpu.SemaphoreType.DMA,
        pltpu.SemaphoreType.DMA,
        pltpu.SemaphoreType.DMA,
    ],
)


def kernel(vertices, faces):
    verts_flat = vertices.reshape(-1)
    faces_pad = jnp.concatenate(
        [faces, jnp.zeros((F_PAD - N_F, 3), jnp.int32)], axis=0).reshape(-1)
    out_flat = _fused_call(verts_flat, faces_pad)
    return out_flat.reshape(5, 2048, 2048)


# speedup vs baseline: 1.0641x; 1.0641x over previous
import jax
import jax.numpy as jnp
from jax import lax
from jax.experimental import pallas as pl
from jax.experimental.pallas import tpu as pltpu
from jax.experimental.pallas import tpu_sc as plsc

N_V = 2048
N_F = 100000
F_PAD = 102400
TILE_F = F_PAD // 16
WIN_F = 160
N_WIN_F = TILE_F // WIN_F
A_BLK = WIN_F * 9
N_ENT = F_PAD * 9
TILE_ENT = N_ENT // 16
WIN_E = 3200
N_WIN_E = TILE_ENT // WIN_E
N_BATCH = WIN_E // 128
Q_SIZE = 512 * 2048
STRIPE = Q_SIZE // 16

_MESH = plsc.VectorSubcoreMesh(
    core_axis_name="c", subcore_axis_name="s", num_cores=2, num_subcores=16)


def _rsqrt(q):
    qc = jnp.maximum(q, jnp.float32(1e-30))
    ib = plsc.bitcast(qc, jnp.int32)
    y = plsc.bitcast(jnp.int32(0x5F3759DF) - (ib >> 1), jnp.float32)
    y = y * (jnp.float32(1.5) - jnp.float32(0.5) * qc * y * y)
    y = y * (jnp.float32(1.5) - jnp.float32(0.5) * qc * y * y)
    return y


def _fused_body(verts_hbm, faces_hbm, out_hbm,
                d_hbm, v_hbm, vtab,
                fwin0, fwin1, dstage0, dstage1, vstage0, vstage1,
                dwin0, dwin1, vwin0, vwin1,
                offst0, offst1, valst0, valst1, zbuf, slab,
                semaf, semao, semin, semstr, semz):
    cc = lax.axis_index("c")
    ss = lax.axis_index("s")
    lane = lax.iota(jnp.int32, 16)
    fwin = (fwin0, fwin1)
    dstage = (dstage0, dstage1)
    vstage = (vstage0, vstage1)
    dwin = (dwin0, dwin1)
    vwin = (vwin0, vwin1)
    offst = (offst0, offst1)
    valst = (valst0, valst1)
    dbase_sc = cc * N_ENT
    vbase_sc = cc * (5 * N_ENT)

    pltpu.sync_copy(verts_hbm, vtab)

    def a_issue_in(w, b):
        fb = ss * TILE_F + w * WIN_F
        pltpu.async_copy(faces_hbm.at[pl.ds(fb * 3, WIN_F * 3)], fwin[b],
                         semaf)

    def a_wait_in(w, b):
        fb = ss * TILE_F + w * WIN_F
        pltpu.make_async_copy(faces_hbm.at[pl.ds(fb * 3, WIN_F * 3)], fwin[b],
                              semaf).wait()

    def a_fire_out(w, b):
        fb = ss * TILE_F + w * WIN_F
        pltpu.async_copy(dstage[b],
                         d_hbm.at[pl.ds(dbase_sc + fb * 9, A_BLK)], semao)
        for m in range(5):
            pltpu.async_copy(
                vstage[b].at[pl.ds(m * A_BLK, A_BLK)],
                v_hbm.at[pl.ds(vbase_sc + m * N_ENT + fb * 9, A_BLK)], semao)

    def a_drain_out(w, b):
        fb = ss * TILE_F + w * WIN_F
        pltpu.make_async_copy(
            dstage[b], d_hbm.at[pl.ds(dbase_sc + fb * 9, A_BLK)],
            semao).wait()
        for m in range(5):
            pltpu.make_async_copy(
                vstage[b].at[pl.ds(m * A_BLK, A_BLK)],
                v_hbm.at[pl.ds(vbase_sc + m * N_ENT + fb * 9, A_BLK)],
                semao).wait()

    a_issue_in(0, 0)
    a_issue_in(1, 1)

    def a_pair(g, _):
        for b in range(2):
            w = g * 2 + b
            a_wait_in(w, b)

            @pl.when(w >= 2)
            def _():
                a_drain_out(w - 2, b)

            def avec(i, b=b, w=w):
                lf = i * 16 + lane
                f0 = plsc.load_gather(fwin[b], [lf * 3])
                f1 = plsc.load_gather(fwin[b], [lf * 3 + 1])
                f2 = plsc.load_gather(fwin[b], [lf * 3 + 2])
                x0 = plsc.load_gather(vtab, [f0 * 3])
                y0 = plsc.load_gather(vtab, [f0 * 3 + 1])
                z0 = plsc.load_gather(vtab, [f0 * 3 + 2])
                x1 = plsc.load_gather(vtab, [f1 * 3])
                y1 = plsc.load_gather(vtab, [f1 * 3 + 1])
                z1 = plsc.load_gather(vtab, [f1 * 3 + 2])
                x2 = plsc.load_gather(vtab, [f2 * 3])
                y2 = plsc.load_gather(vtab, [f2 * 3 + 1])
                z2 = plsc.load_gather(vtab, [f2 * 3 + 2])

                ax, ay, az = x1 - x0, y1 - y0, z1 - z0
                bx, by, bz = x2 - x0, y2 - y0, z2 - z0
                gx, gy, gz = x2 - x1, y2 - y1, z2 - z1
                cxv = ay * bz - az * by
                cyv = az * bx - ax * bz
                czv = ax * by - ay * bx
                q = cxv * cxv + cyv * cyv + czv * czv
                inv = _rsqrt(q)
                d0 = ax * bx + ay * by + az * bz
                d1 = gx * (-ax) + gy * (-ay) + gz * (-az)
                d2 = bx * gx + by * gy + bz * gz
                c0 = d0 * inv
                c1 = d1 * inv
                c2 = d2 * inv
                area = jnp.float32(0.5) * q * inv

                h = jnp.float32(0.5)
                svals = (h * (c1 + c2), h * (c0 + c2), h * (c0 + c1),
                         -h * c2, -h * c2, -h * c0, -h * c0, -h * c1, -h * c1)
                d6 = area * jnp.float32(1.0 / 6.0)
                o12 = area * jnp.float32(1.0 / 12.0)
                mvals = (d6, d6, d6, o12, o12, o12, o12, o12, o12)
                a30 = area * jnp.float32(1.0 / 30.0)
                a60 = area * jnp.float32(1.0 / 60.0)

                rows = (f0, f1, f2, f0, f1, f1, f2, f2, f0)
                cols = (f0, f1, f2, f1, f0, f2, f1, f0, f2)
                gf = ss * TILE_F + w * WIN_F + lf
                real = gf < N_F

                pos = []
                for u0, u1, u2 in ((x0, x1, x2), (y0, y1, y2), (z0, z1, z2)):
                    v00 = a30 * (jnp.float32(3.0) * u0 + u1 + u2)
                    v11 = a30 * (u0 + jnp.float32(3.0) * u1 + u2)
                    v22 = a30 * (u0 + u1 + jnp.float32(3.0) * u2)
                    v01 = a60 * (jnp.float32(2.0) * (u0 + u1) + u2)
                    v12 = a60 * (u0 + jnp.float32(2.0) * (u1 + u2))
                    v02 = a60 * (jnp.float32(2.0) * (u0 + u2) + u1)
                    pos.append((v00, v11, v22, v01, v01, v12, v12, v02, v02))

                zero = jnp.float32(0.0)
                for k in range(9):
                    dk = jnp.where(real, rows[k] * 2048 + cols[k], gf * 9 + k)
                    dstage[b][pl.ds(k * WIN_F + i * 16, 16)] = dk
                    allv = (svals[k], mvals[k], pos[0][k], pos[1][k],
                            pos[2][k])
                    for m in range(5):
                        vk = jnp.where(real, allv[m], zero)
                        vstage[b][pl.ds(m * A_BLK + k * WIN_F + i * 16,
                                        16)] = vk

            plsc.parallel_loop(0, WIN_F // 16)(avec)

            @pl.when(w + 2 < N_WIN_F)
            def _():
                a_issue_in(w + 2, b)

            a_fire_out(w, b)
        return 0

    lax.fori_loop(0, N_WIN_F // 2, a_pair, 0)
    a_drain_out(N_WIN_F - 2, 0)
    a_drain_out(N_WIN_F - 1, 1)
    plsc.subcore_barrier()

    ebase = ss * TILE_ENT

    def zinit(i, _):
        zbuf[pl.ds(i * 16, 16)] = jnp.zeros((16,), jnp.float32)
        return 0
    lax.fori_loop(0, 2048 // 16, zinit, 0)

    def issue_in(m, w, b):
        pltpu.async_copy(
            d_hbm.at[pl.ds(dbase_sc + ebase + w * WIN_E, WIN_E)],
            dwin[b], semin)
        pltpu.async_copy(
            v_hbm.at[pl.ds(vbase_sc + m * N_ENT + ebase + w * WIN_E, WIN_E)],
            vwin[b], semin)

    def wait_in(m, w, b):
        pltpu.make_async_copy(
            d_hbm.at[pl.ds(dbase_sc + ebase + w * WIN_E, WIN_E)],
            dwin[b], semin).wait()
        pltpu.make_async_copy(
            v_hbm.at[pl.ds(vbase_sc + m * N_ENT + ebase + w * WIN_E, WIN_E)],
            vwin[b], semin).wait()

    def fire_streams(b):
        for j in range(N_BATCH):
            pltpu.async_copy(valst[b].at[j], slab.at[offst[b].at[j]], semstr,
                             add=True)

    def drain_streams(b):
        for j in range(N_BATCH):
            pltpu.make_async_copy(valst[b].at[j], slab.at[offst[b].at[j]],
                                  semstr).wait()

    def pass_body(p, _):
        m = p // 2
        quarter = (p % 2) * 2 + cc
        lo = quarter * Q_SIZE

        zcps = [pltpu.async_copy(
            zbuf, slab.at[pl.ds(ss * STRIPE + j * 2048, 2048)], semz)
            for j in range(STRIPE // 2048)]
        for cp in zcps:
            cp.wait()
        plsc.subcore_barrier()

        issue_in(m, 0, 0)
        issue_in(m, 1, 1)

        def gpair(g, _):
            for b in range(2):
                w = g * 2 + b
                wait_in(m, w, b)

                @pl.when(w >= 2)
                def _():
                    drain_streams(b)

                def bvec(i, b=b, w=w):
                    dd = dwin[b][pl.ds(i * 16, 16)]
                    vv = vwin[b][pl.ds(i * 16, 16)]
                    dq = dd - lo
                    msk = plsc.bitcast(dq, jnp.uint32) < jnp.uint32(Q_SIZE)
                    off = jnp.where(msk, dq, w * WIN_E + i * 16 + lane)
                    val = jnp.where(msk, vv, jnp.float32(0.0))
                    offst[b][i >> 3, pl.ds((i & 7) * 16, 16)] = off
                    valst[b][i >> 3, pl.ds((i & 7) * 16, 16)] = val

                plsc.parallel_loop(0, WIN_E // 16, unroll=4)(bvec)

                @pl.when(w + 2 < N_WIN_E)
                def _():
                    issue_in(m, w + 2, b)

                fire_streams(b)
            return 0

        lax.fori_loop(0, N_WIN_E // 2, gpair, 0)
        drain_streams(0)
        drain_streams(1)
        plsc.subcore_barrier()
        pltpu.sync_copy(
            slab.at[pl.ds(ss * STRIPE, STRIPE)],
            out_hbm.at[pl.ds(m * (2048 * 2048) + quarter * Q_SIZE + ss * STRIPE,
                             STRIPE)])
        return 0

    lax.fori_loop(0, 10, pass_body, 0)


_fused_call = pl.kernel(
    _fused_body,
    out_type=jax.ShapeDtypeStruct((5 * 2048 * 2048,), jnp.float32),
    mesh=_MESH,
    compiler_params=pltpu.CompilerParams(needs_layout_passes=False),
    scratch_types=[
        pltpu.HBM((2 * N_ENT,), jnp.int32),
        pltpu.HBM((2 * 5 * N_ENT,), jnp.float32),
        pltpu.VMEM((N_V * 3,), jnp.float32),
        pltpu.VMEM((WIN_F * 3,), jnp.int32),
        pltpu.VMEM((WIN_F * 3,), jnp.int32),
        pltpu.VMEM((A_BLK,), jnp.int32),
        pltpu.VMEM((A_BLK,), jnp.int32),
        pltpu.VMEM((5 * A_BLK,), jnp.float32),
        pltpu.VMEM((5 * A_BLK,), jnp.float32),
        pltpu.VMEM((WIN_E,), jnp.int32),
        pltpu.VMEM((WIN_E,), jnp.int32),
        pltpu.VMEM((WIN_E,), jnp.float32),
        pltpu.VMEM((WIN_E,), jnp.float32),
        pltpu.VMEM((N_BATCH, 128), jnp.int32),
        pltpu.VMEM((N_BATCH, 128), jnp.int32),
        pltpu.VMEM((N_BATCH, 128), jnp.float32),
        pltpu.VMEM((N_BATCH, 128), jnp.float32),
        pltpu.VMEM((2048,), jnp.float32),
        pltpu.VMEM_SHARED((Q_SIZE,), jnp.float32),
        pltpu.SemaphoreType.DMA,
        pltpu.SemaphoreType.DMA,
        pltpu.SemaphoreType.DMA,
        pltpu.SemaphoreType.DMA,
        pltpu.SemaphoreType.DMA,
    ],
)


def kernel(vertices, faces):
    verts_flat = vertices.reshape(-1)
    faces_pad = jnp.concatenate(
        [faces, jnp.zeros((F_PAD - N_F, 3), jnp.int32)], axis=0).reshape(-1)
    out_flat = _fused_call(verts_flat, faces_pad)
    return out_flat.reshape(5, 2048, 2048)
